# baseline (speedup 1.0000x reference)
import functools
import jax
import jax.numpy as jnp
from jax import lax
from jax.experimental import pallas as pl
from jax.experimental.pallas import tpu as pltpu
from jax.experimental.pallas import tpu_sc as plsc

IN_FEATS = 256
OUT_FEATS = 256
N_USERS = 25000
N_ITEMS = 25000
N_EDGES = 1600000

NC, NS, L = 2, 16, 16
NW = NC * NS
NPAD = 25088
GB = 80
ACC_ROWS = 25600
FCH = 4
CW = OUT_FEATS // FCH

_mesh = plsc.VectorSubcoreMesh(
    core_axis_name="c", subcore_axis_name="s", num_cores=NC, num_subcores=NS)


NG_ROWS = 2 * NPAD
NG_PW = NG_ROWS // NW
NG_BLK = 112


@functools.partial(
    pl.kernel,
    out_type=jax.ShapeDtypeStruct((NG_ROWS, IN_FEATS), jnp.float32),
    mesh=_mesh,
    compiler_params=pltpu.CompilerParams(use_tc_tiling_on_sc=False,
                                         needs_layout_passes=False),
    scratch_types=[
        pltpu.VMEM((NG_BLK,), jnp.int32),
        pltpu.VMEM((NG_BLK, IN_FEATS), jnp.float32),
        pltpu.SemaphoreType.DMA,
    ],
)
def _node_gather(feat_hbm, ids_hbm, out_hbm, idx_v, rows_v, sem):
    wid = lax.axis_index("s") * NC + lax.axis_index("c")
    base = wid * NG_PW
    for k in range(NG_PW // NG_BLK):
        off = base + k * NG_BLK
        pltpu.sync_copy(ids_hbm.at[pl.ds(off, NG_BLK)], idx_v)
        pltpu.async_copy(feat_hbm.at[idx_v], rows_v, sem).wait()
        pltpu.sync_copy(rows_v, out_hbm.at[pl.ds(off, NG_BLK)])


EPW = N_EDGES // NW
LG_MEGA = 2000
LG_NSB = LG_MEGA // GB
LG_NMEGA = EPW // LG_MEGA


@functools.partial(
    pl.kernel,
    out_type=jax.ShapeDtypeStruct((N_EDGES,), jnp.float32),
    mesh=_mesh,
    compiler_params=pltpu.CompilerParams(use_tc_tiling_on_sc=False,
                                         needs_layout_passes=False),
    scratch_types=[
        pltpu.VMEM((LG_MEGA,), jnp.int32),
        pltpu.VMEM((LG_MEGA,), jnp.int32),
        pltpu.VMEM((LG_MEGA,), jnp.float32),
        pltpu.VMEM((2, GB, IN_FEATS), jnp.float32),
        pltpu.VMEM((2, GB, IN_FEATS), jnp.float32),
        pltpu.SemaphoreType.DMA,
        pltpu.SemaphoreType.DMA,
        pltpu.SemaphoreType.DMA,
        pltpu.SemaphoreType.DMA,
    ],
)
def _edge_logits(h_hbm, esrc_hbm, edst_hbm, out_hbm,
                 es_v, ed_v, lg_v, u_v, v_v, su0, su1, sv0, sv1):
    wid = lax.axis_index("s") * NC + lax.axis_index("c")
    base = wid * EPW
    sem_u = (su0, su1)
    sem_v = (sv0, sv1)

    def fire(k, slot):
        cu = pltpu.async_copy(
            h_hbm.at[es_v.at[pl.ds(k * GB, GB)]], u_v.at[slot], sem_u[slot])
        cv = pltpu.async_copy(
            h_hbm.at[ed_v.at[pl.ds(k * GB, GB)]], v_v.at[slot], sem_v[slot])
        return cu, cv

    def mega(m, carry):
        off = base + m * LG_MEGA
        pltpu.sync_copy(esrc_hbm.at[pl.ds(off, LG_MEGA)], es_v)
        pltpu.sync_copy(edst_hbm.at[pl.ds(off, LG_MEGA)], ed_v)

        def obody(q, c2):
            ed_v[pl.ds(q * L, L)] = ed_v[pl.ds(q * L, L)] + NPAD
            return c2
        lax.fori_loop(0, LG_MEGA // L, obody, 0)

        pend = [None, None]
        pend[0] = fire(0, 0)
        for k in range(LG_NSB):
            slot = k % 2
            if k + 1 < LG_NSB:
                pend[1 - slot] = fire(k + 1, 1 - slot)
            cu, cv = pend[slot]
            cu.wait()
            cv.wait()

            iot = lax.iota(jnp.int32, L)

            def group(g, c3):
                def jbody(j, res):
                    e = g * L + j
                    accs = []
                    for a0 in range(4):
                        c0 = a0 * 4
                        acc = (u_v[slot, e, pl.ds(c0 * L, L)]
                               * v_v[slot, e, pl.ds(c0 * L, L)])
                        for i in range(1, 4):
                            acc = acc + (u_v[slot, e, pl.ds((c0 + i) * L, L)]
                                         * v_v[slot, e, pl.ds((c0 + i) * L, L)])
                        accs.append(acc)
                    tot = (accs[0] + accs[1]) + (accs[2] + accs[3])
                    s = jnp.sum(tot) * 0.0625
                    return jnp.where(iot == j, s, res)

                res = lax.fori_loop(0, L, jbody, jnp.zeros((L,), jnp.float32))
                lg_v[pl.ds(k * GB + g * L, L)] = res
                return c3

            lax.fori_loop(0, GB // L, group, 0)
        pltpu.sync_copy(lg_v, out_hbm.at[pl.ds(off, LG_MEGA)])
        return carry

    lax.fori_loop(0, LG_NMEGA, mega, 0)


SM_COLS = 128
SM_ROWS = N_EDGES // SM_COLS
SM_BM = 500


def _sm_max_body(x_ref, o_ref):
    @pl.when(pl.program_id(0) == 0)
    def _():
        o_ref[0, 0] = -jnp.inf

    o_ref[0, 0] = jnp.maximum(o_ref[0, 0], jnp.max(x_ref[...]))


def _sm_exp_body(x_ref, m_ref, w_ref, z_ref):
    @pl.when(pl.program_id(0) == 0)
    def _():
        z_ref[0, 0] = 0.0

    w = jnp.exp(x_ref[...] - m_ref[0, 0])
    w_ref[...] = w
    z_ref[0, 0] += jnp.sum(w)


def _sm_scale_body(w_ref, z_ref, a_ref):
    a_ref[...] = w_ref[...] * (1.0 / z_ref[0, 0])


def _softmax(logits):
    x = logits.reshape(SM_ROWS, SM_COLS)
    m = pl.pallas_call(
        _sm_max_body,
        grid=(1,),
        in_specs=[pl.BlockSpec((SM_ROWS, SM_COLS), lambda i: (0, 0))],
        out_specs=pl.BlockSpec(memory_space=pltpu.SMEM),
        out_shape=jax.ShapeDtypeStruct((1, 1), jnp.float32),
    )(x)
    w, z = pl.pallas_call(
        _sm_exp_body,
        grid=(1,),
        in_specs=[
            pl.BlockSpec((SM_ROWS, SM_COLS), lambda i: (0, 0)),
            pl.BlockSpec(memory_space=pltpu.SMEM),
        ],
        out_specs=[
            pl.BlockSpec((SM_ROWS, SM_COLS), lambda i: (0, 0)),
            pl.BlockSpec(memory_space=pltpu.SMEM),
        ],
        out_shape=[
            jax.ShapeDtypeStruct((SM_ROWS, SM_COLS), jnp.float32),
            jax.ShapeDtypeStruct((1, 1), jnp.float32),
        ],
    )(x, m)
    a = pl.pallas_call(
        _sm_scale_body,
        grid=(1,),
        in_specs=[
            pl.BlockSpec((SM_ROWS, SM_COLS), lambda i: (0, 0)),
            pl.BlockSpec(memory_space=pltpu.SMEM),
        ],
        out_specs=pl.BlockSpec((SM_ROWS, SM_COLS), lambda i: (0, 0)),
        out_shape=jax.ShapeDtypeStruct((SM_ROWS, SM_COLS), jnp.float32),
    )(w, z)
    return a.reshape(N_EDGES)


TR_BM = 784


def _tr_body(h_ref, w_ref, b_ref, o_ref):
    o_ref[0] = jax.nn.relu(
        lax.dot_general(h_ref[...], w_ref[0], (((1,), (1,)), ((), ())),
                        preferred_element_type=jnp.float32)
        + b_ref[0]).astype(jnp.bfloat16)


def _transform_chunks(h_all, W, b, blk_off):
    nb = NPAD // TR_BM
    return pl.pallas_call(
        _tr_body,
        grid=(nb, FCH),
        in_specs=[
            pl.BlockSpec((TR_BM, IN_FEATS), lambda i, c: (i + blk_off, 0)),
            pl.BlockSpec((1, CW, IN_FEATS), lambda i, c: (c, 0, 0)),
            pl.BlockSpec((1, 1, CW), lambda i, c: (c, 0, 0)),
        ],
        out_specs=pl.BlockSpec((1, TR_BM, CW), lambda i, c: (c, i, 0)),
        out_shape=jax.ShapeDtypeStruct((FCH, NPAD, CW), jnp.bfloat16),
    )(h_all, W.reshape(FCH, CW, IN_FEATS), b.reshape(FCH, 1, CW))


EPS = N_EDGES // NS
ZROWS = ACC_ROWS // NS
AG_MEGA = 4000
AG_NSB = AG_MEGA // GB
AG_NMEGA = EPS // AG_MEGA
NSLOT = 8
AG_PF = 3
CW2 = 32
NCH2 = OUT_FEATS // CW2
NPH = 2 * (NCH2 // NC)


@functools.partial(
    pl.kernel,
    out_type=jax.ShapeDtypeStruct((2 * NCH2 * ACC_ROWS, CW2), jnp.float32),
    mesh=_mesh,
    compiler_params=pltpu.CompilerParams(use_tc_tiling_on_sc=False,
                                         needs_layout_passes=False),
    scratch_types=[
        pltpu.VMEM((AG_MEGA,), jnp.int32),
        pltpu.VMEM((AG_NSB, 1, GB), jnp.int32),
        pltpu.VMEM((AG_MEGA + L,), jnp.float32),
        pltpu.VMEM((NSLOT, GB, CW2), jnp.bfloat16),
        pltpu.VMEM((NSLOT, GB, CW2), jnp.float32),
        pltpu.VMEM_SHARED((ACC_ROWS, CW2), jnp.float32),
    ] + [pltpu.SemaphoreType.DMA] * (2 * NSLOT + 3),
)
def _aggregate(table_hbm, idsg_hbm, idss_hbm, alpha_hbm, zeros_hbm,
               out_hbm, gi_v, si_v, al_v, rowsbf_v, rows_v, acc_sh, *sems):
    cid = lax.axis_index("c")
    sid = lax.axis_index("s")
    ebase = sid * EPS
    sem_g = sems[:NSLOT]
    sem_s = sems[NSLOT:2 * NSLOT]
    sem_sc = sems[2 * NSLOT:]

    def phase(p, pcarry):
        dirn = p // (NPH // 2)
        lchunk = p % (NPH // 2)
        cc = cid * (NPH // 2) + lchunk
        goffc = (cc % 2) + (dirn * (NCH2 // 2) + cc // 2) * (2 * NPAD)

        pltpu.sync_copy(zeros_hbm, acc_sh.at[pl.ds(sid * ZROWS, ZROWS)])
        plsc.subcore_barrier()

        def mega(m, carry):
            off = ebase + m * AG_MEGA
            c1 = pltpu.async_copy(
                idsg_hbm.at[pl.ds(dirn * N_EDGES + off, AG_MEGA)], gi_v,
                sem_sc[0])
            c2 = pltpu.async_copy(
                idss_hbm.at[pl.ds(dirn * (N_EDGES // GB) + off // GB, AG_NSB)],
                si_v, sem_sc[1])
            c3 = pltpu.async_copy(alpha_hbm.at[pl.ds(off, AG_MEGA)],
                                  al_v.at[pl.ds(0, AG_MEGA)], sem_sc[2])
            c1.wait()
            c2.wait()
            c3.wait()

            def obody(q, c2):
                gi_v[pl.ds(q * L, L)] = gi_v[pl.ds(q * L, L)] * 2 + goffc
                return c2
            lax.fori_loop(0, AG_MEGA // L, obody, 0)

            pend_g = [None] * NSLOT
            pend_s = [None] * NSLOT

            def fire(k):
                slot = k % NSLOT
                pend_g[slot] = pltpu.async_copy(
                    table_hbm.at[gi_v.at[pl.ds(k * GB, GB)]],
                    rowsbf_v.at[slot], sem_g[slot])

            for kp in range(AG_PF):
                fire(kp)
            for k in range(AG_NSB):
                slot = k % NSLOT
                if k + AG_PF < AG_NSB:
                    nslot = (k + AG_PF) % NSLOT
                    if pend_s[nslot] is not None:
                        pend_s[nslot].wait()
                        pend_s[nslot] = None
                    fire(k + AG_PF)
                pend_g[slot].wait()
                rows = rows_v.at[slot]

                def ebody(gq, c3):
                    a16 = al_v[pl.ds(k * GB + gq * 8, L)]
                    for j in range(8):
                        e = gq * 8 + j
                        av = jnp.full((L,), a16[j])
                        pa, pb = plsc.unpack(
                            rowsbf_v[slot, e, :],
                            format=plsc.PackFormat.INTERLEAVED)
                        rows_v[slot, e, pl.ds(0, L)] = pa * av
                        rows_v[slot, e, pl.ds(L, L)] = pb * av
                    return c3

                lax.fori_loop(0, GB // 8, ebody, 0)
                pend_s[slot] = pltpu.async_copy(
                    rows, acc_sh.at[si_v.at[k, 0]], sem_s[slot], add=True)
            for pd in pend_s:
                if pd is not None:
                    pd.wait()
            return carry

        lax.fori_loop(0, AG_NMEGA, mega, 0)
        plsc.subcore_barrier()
        pltpu.sync_copy(
            acc_sh.at[pl.ds(sid * ZROWS, ZROWS)],
            out_hbm.at[pl.ds((dirn * NCH2 + cc) * ACC_ROWS + sid * ZROWS,
                             ZROWS)])
        plsc.subcore_barrier()
        return pcarry

    lax.fori_loop(0, NPH, phase, 0)


def kernel(feat, user_ids, item_ids, edge_src, edge_dst,
           W_src, b_src, W_dst, b_dst):
    user_ids = user_ids.astype(jnp.int32)
    item_ids = item_ids.astype(jnp.int32)
    edge_src = edge_src.astype(jnp.int32)
    edge_dst = edge_dst.astype(jnp.int32)

    pad = jnp.zeros((NPAD - N_USERS,), jnp.int32)
    ids_all = jnp.concatenate([user_ids, pad, item_ids, pad])

    h_all = _node_gather(feat, ids_all)
    logits = _edge_logits(h_all, edge_src, edge_dst)
    alpha = _softmax(logits)

    idxp = jnp.arange(OUT_FEATS)
    srcp = (idxp // CW2) * CW2 + (idxp % 2) * L + (idxp % CW2) // 2
    fsrc = _transform_chunks(h_all, W_src[srcp], b_src[srcp], 0).reshape(
        -1, CW2)
    fdst = _transform_chunks(h_all, W_dst[srcp], b_dst[srcp],
                             NPAD // TR_BM).reshape(-1, CW2)
    table = jnp.concatenate([fsrc, fdst], axis=0)

    idsg = jnp.concatenate([edge_src, edge_dst])
    idss = jnp.concatenate([edge_dst, edge_src]).reshape(-1, 1, GB)
    zeros = jnp.zeros((ZROWS, CW2), jnp.float32)
    out = _aggregate(table, idsg, idss, alpha, zeros)

    parts = out.reshape(2, NCH2, ACC_ROWS, CW2)
    e_new_item = (parts[0, :, :N_ITEMS]
                  .transpose(1, 0, 2).reshape(N_ITEMS, OUT_FEATS))
    e_new_user = (parts[1, :, :N_USERS]
                  .transpose(1, 0, 2).reshape(N_USERS, OUT_FEATS))
    return jnp.concatenate([e_new_user, e_new_item], axis=0)


# speedup vs baseline: 1.4055x; 1.4055x over previous
import functools
import jax
import jax.numpy as jnp
from jax import lax
from jax.experimental import pallas as pl
from jax.experimental.pallas import tpu as pltpu
from jax.experimental.pallas import tpu_sc as plsc

IN_FEATS = 256
OUT_FEATS = 256
N_USERS = 25000
N_ITEMS = 25000
N_EDGES = 1600000

NC, NS, L = 2, 16, 16
NW = NC * NS
NPAD = 25088
GB = 80
ACC_ROWS = 25600
FCH = 4
CW = OUT_FEATS // FCH

_mesh = plsc.VectorSubcoreMesh(
    core_axis_name="c", subcore_axis_name="s", num_cores=NC, num_subcores=NS)


NG_ROWS = 2 * NPAD
NG_PW = NG_ROWS // NW
NG_BLK = 112


@functools.partial(
    pl.kernel,
    out_type=jax.ShapeDtypeStruct((NG_ROWS, IN_FEATS), jnp.float32),
    mesh=_mesh,
    compiler_params=pltpu.CompilerParams(use_tc_tiling_on_sc=False,
                                         needs_layout_passes=False),
    scratch_types=[
        pltpu.VMEM((NG_BLK,), jnp.int32),
        pltpu.VMEM((NG_BLK, IN_FEATS), jnp.float32),
        pltpu.SemaphoreType.DMA,
    ],
)
def _node_gather(feat_hbm, ids_hbm, out_hbm, idx_v, rows_v, sem):
    wid = lax.axis_index("s") * NC + lax.axis_index("c")
    base = wid * NG_PW
    for k in range(NG_PW // NG_BLK):
        off = base + k * NG_BLK
        pltpu.sync_copy(ids_hbm.at[pl.ds(off, NG_BLK)], idx_v)
        pltpu.async_copy(feat_hbm.at[idx_v], rows_v, sem).wait()
        pltpu.sync_copy(rows_v, out_hbm.at[pl.ds(off, NG_BLK)])


EPW = N_EDGES // NW
LG_MEGA = 2000
LG_NSB = LG_MEGA // GB
LG_NMEGA = EPW // LG_MEGA


@functools.partial(
    pl.kernel,
    out_type=jax.ShapeDtypeStruct((N_EDGES,), jnp.float32),
    mesh=_mesh,
    compiler_params=pltpu.CompilerParams(use_tc_tiling_on_sc=False,
                                         needs_layout_passes=False),
    scratch_types=[
        pltpu.VMEM((LG_MEGA,), jnp.int32),
        pltpu.VMEM((LG_MEGA,), jnp.int32),
        pltpu.VMEM((LG_MEGA,), jnp.float32),
        pltpu.VMEM((2, GB, IN_FEATS), jnp.float32),
        pltpu.VMEM((2, GB, IN_FEATS), jnp.float32),
        pltpu.SemaphoreType.DMA,
        pltpu.SemaphoreType.DMA,
        pltpu.SemaphoreType.DMA,
        pltpu.SemaphoreType.DMA,
    ],
)
def _edge_logits(h_hbm, esrc_hbm, edst_hbm, out_hbm,
                 es_v, ed_v, lg_v, u_v, v_v, su0, su1, sv0, sv1):
    wid = lax.axis_index("s") * NC + lax.axis_index("c")
    base = wid * EPW
    sem_u = (su0, su1)
    sem_v = (sv0, sv1)

    def fire(k, slot):
        cu = pltpu.async_copy(
            h_hbm.at[es_v.at[pl.ds(k * GB, GB)]], u_v.at[slot], sem_u[slot])
        cv = pltpu.async_copy(
            h_hbm.at[ed_v.at[pl.ds(k * GB, GB)]], v_v.at[slot], sem_v[slot])
        return cu, cv

    def mega(m, carry):
        off = base + m * LG_MEGA
        pltpu.sync_copy(esrc_hbm.at[pl.ds(off, LG_MEGA)], es_v)
        pltpu.sync_copy(edst_hbm.at[pl.ds(off, LG_MEGA)], ed_v)

        def obody(q, c2):
            ed_v[pl.ds(q * L, L)] = ed_v[pl.ds(q * L, L)] + NPAD
            return c2
        lax.fori_loop(0, LG_MEGA // L, obody, 0)

        pend = [None, None]
        pend[0] = fire(0, 0)
        for k in range(LG_NSB):
            slot = k % 2
            if k + 1 < LG_NSB:
                pend[1 - slot] = fire(k + 1, 1 - slot)
            cu, cv = pend[slot]
            cu.wait()
            cv.wait()

            iot = lax.iota(jnp.int32, L)

            def group(g, c3):
                def jbody(j, res):
                    e = g * L + j
                    accs = []
                    for a0 in range(4):
                        c0 = a0 * 4
                        acc = (u_v[slot, e, pl.ds(c0 * L, L)]
                               * v_v[slot, e, pl.ds(c0 * L, L)])
                        for i in range(1, 4):
                            acc = acc + (u_v[slot, e, pl.ds((c0 + i) * L, L)]
                                         * v_v[slot, e, pl.ds((c0 + i) * L, L)])
                        accs.append(acc)
                    tot = (accs[0] + accs[1]) + (accs[2] + accs[3])
                    s = jnp.sum(tot) * 0.0625
                    return jnp.where(iot == j, s, res)

                res = lax.fori_loop(0, L, jbody, jnp.zeros((L,), jnp.float32))
                lg_v[pl.ds(k * GB + g * L, L)] = res
                return c3

            lax.fori_loop(0, GB // L, group, 0)
        pltpu.sync_copy(lg_v, out_hbm.at[pl.ds(off, LG_MEGA)])
        return carry

    lax.fori_loop(0, LG_NMEGA, mega, 0)


SM_COLS = 128
SM_ROWS = N_EDGES // SM_COLS
SM_BM = 500


def _sm_max_body(x_ref, o_ref):
    @pl.when(pl.program_id(0) == 0)
    def _():
        o_ref[0, 0] = -jnp.inf

    o_ref[0, 0] = jnp.maximum(o_ref[0, 0], jnp.max(x_ref[...]))


def _sm_exp_body(x_ref, m_ref, w_ref, z_ref):
    @pl.when(pl.program_id(0) == 0)
    def _():
        z_ref[0, 0] = 0.0

    w = jnp.exp(x_ref[...] - m_ref[0, 0])
    w_ref[...] = w
    z_ref[0, 0] += jnp.sum(w)


def _sm_scale_body(w_ref, z_ref, a_ref):
    a_ref[...] = w_ref[...] * (1.0 / z_ref[0, 0])


def _softmax(logits):
    x = logits.reshape(SM_ROWS, SM_COLS)
    m = pl.pallas_call(
        _sm_max_body,
        grid=(1,),
        in_specs=[pl.BlockSpec((SM_ROWS, SM_COLS), lambda i: (0, 0))],
        out_specs=pl.BlockSpec(memory_space=pltpu.SMEM),
        out_shape=jax.ShapeDtypeStruct((1, 1), jnp.float32),
    )(x)
    w, z = pl.pallas_call(
        _sm_exp_body,
        grid=(1,),
        in_specs=[
            pl.BlockSpec((SM_ROWS, SM_COLS), lambda i: (0, 0)),
            pl.BlockSpec(memory_space=pltpu.SMEM),
        ],
        out_specs=[
            pl.BlockSpec((SM_ROWS, SM_COLS), lambda i: (0, 0)),
            pl.BlockSpec(memory_space=pltpu.SMEM),
        ],
        out_shape=[
            jax.ShapeDtypeStruct((SM_ROWS, SM_COLS), jnp.float32),
            jax.ShapeDtypeStruct((1, 1), jnp.float32),
        ],
    )(x, m)
    a = pl.pallas_call(
        _sm_scale_body,
        grid=(1,),
        in_specs=[
            pl.BlockSpec((SM_ROWS, SM_COLS), lambda i: (0, 0)),
            pl.BlockSpec(memory_space=pltpu.SMEM),
        ],
        out_specs=pl.BlockSpec((SM_ROWS, SM_COLS), lambda i: (0, 0)),
        out_shape=jax.ShapeDtypeStruct((SM_ROWS, SM_COLS), jnp.float32),
    )(w, z)
    return a.reshape(N_EDGES)


TR_BM = 784


def _tr_body(h_ref, w_ref, b_ref, o_ref):
    o_ref[0] = jax.nn.relu(
        lax.dot_general(h_ref[...], w_ref[0], (((1,), (1,)), ((), ())),
                        preferred_element_type=jnp.float32) + b_ref[0])


def _transform_chunks(h_all, W, b, blk_off):
    nb = NPAD // TR_BM
    return pl.pallas_call(
        _tr_body,
        grid=(nb, FCH),
        in_specs=[
            pl.BlockSpec((TR_BM, IN_FEATS), lambda i, c: (i + blk_off, 0)),
            pl.BlockSpec((1, CW, IN_FEATS), lambda i, c: (c, 0, 0)),
            pl.BlockSpec((1, 1, CW), lambda i, c: (c, 0, 0)),
        ],
        out_specs=pl.BlockSpec((1, TR_BM, CW), lambda i, c: (c, i, 0)),
        out_shape=jax.ShapeDtypeStruct((FCH, NPAD, CW), jnp.float32),
    )(h_all, W.reshape(FCH, CW, IN_FEATS), b.reshape(FCH, 1, CW))


EPS = N_EDGES // NS
ZROWS = ACC_ROWS // NS
AG_MEGA = 4000
AG_NSB = AG_MEGA // GB
AG_NMEGA = EPS // AG_MEGA
NSLOT = 8
AG_PF = 3
CW2 = 32
NCH2 = OUT_FEATS // CW2
NPH = 2 * (NCH2 // NC)


@functools.partial(
    pl.kernel,
    out_type=jax.ShapeDtypeStruct((2 * NCH2 * ACC_ROWS, CW2), jnp.float32),
    mesh=_mesh,
    compiler_params=pltpu.CompilerParams(use_tc_tiling_on_sc=False,
                                         needs_layout_passes=False),
    scratch_types=[
        pltpu.VMEM((AG_MEGA,), jnp.int32),
        pltpu.VMEM((AG_NSB, 1, GB), jnp.int32),
        pltpu.VMEM((AG_MEGA + L,), jnp.float32),
        pltpu.VMEM((NSLOT, GB, CW2), jnp.float32),
        pltpu.VMEM_SHARED((ACC_ROWS, CW2), jnp.float32),
    ] + [pltpu.SemaphoreType.DMA] * (2 * NSLOT + 3),
)
def _aggregate(table_hbm, idsg_hbm, idss_hbm, alpha_hbm, zeros_hbm,
               out_hbm, gi_v, si_v, al_v, rows_v, acc_sh, *sems):
    cid = lax.axis_index("c")
    sid = lax.axis_index("s")
    ebase = sid * EPS
    sem_g = sems[:NSLOT]
    sem_s = sems[NSLOT:2 * NSLOT]
    sem_sc = sems[2 * NSLOT:]

    def phase(p, pcarry):
        dirn = p // (NPH // 2)
        lchunk = p % (NPH // 2)
        cc = cid * (NPH // 2) + lchunk
        goffc = (cc % 2) + (dirn * (NCH2 // 2) + cc // 2) * (2 * NPAD)

        pltpu.sync_copy(zeros_hbm, acc_sh.at[pl.ds(sid * ZROWS, ZROWS)])
        plsc.subcore_barrier()

        def mega(m, carry):
            off = ebase + m * AG_MEGA
            c1 = pltpu.async_copy(
                idsg_hbm.at[pl.ds(dirn * N_EDGES + off, AG_MEGA)], gi_v,
                sem_sc[0])
            c2 = pltpu.async_copy(
                idss_hbm.at[pl.ds(dirn * (N_EDGES // GB) + off // GB, AG_NSB)],
                si_v, sem_sc[1])
            c3 = pltpu.async_copy(alpha_hbm.at[pl.ds(off, AG_MEGA)],
                                  al_v.at[pl.ds(0, AG_MEGA)], sem_sc[2])
            c1.wait()
            c2.wait()
            c3.wait()

            def obody(q, c2):
                gi_v[pl.ds(q * L, L)] = gi_v[pl.ds(q * L, L)] * 2 + goffc
                return c2
            lax.fori_loop(0, AG_MEGA // L, obody, 0)

            pend_g = [None] * NSLOT
            pend_s = [None] * NSLOT

            def fire(k):
                slot = k % NSLOT
                pend_g[slot] = pltpu.async_copy(
                    table_hbm.at[gi_v.at[pl.ds(k * GB, GB)]],
                    rows_v.at[slot], sem_g[slot])

            for kp in range(AG_PF):
                fire(kp)
            for k in range(AG_NSB):
                slot = k % NSLOT
                if k + AG_PF < AG_NSB:
                    nslot = (k + AG_PF) % NSLOT
                    if pend_s[nslot] is not None:
                        pend_s[nslot].wait()
                        pend_s[nslot] = None
                    fire(k + AG_PF)
                pend_g[slot].wait()
                rows = rows_v.at[slot]

                def ebody(gq, c3):
                    a16 = al_v[pl.ds(k * GB + gq * L, L)]
                    for j in range(L):
                        e = gq * L + j
                        av = jnp.full((L,), a16[j])
                        for c2 in range(CW2 // L):
                            sl = pl.ds(c2 * L, L)
                            rows_v[slot, e, sl] = rows_v[slot, e, sl] * av
                    return c3

                lax.fori_loop(0, GB // L, ebody, 0)
                pend_s[slot] = pltpu.async_copy(
                    rows, acc_sh.at[si_v.at[k, 0]], sem_s[slot], add=True)
            for pd in pend_s:
                if pd is not None:
                    pd.wait()
            return carry

        lax.fori_loop(0, AG_NMEGA, mega, 0)
        plsc.subcore_barrier()
        pltpu.sync_copy(
            acc_sh.at[pl.ds(sid * ZROWS, ZROWS)],
            out_hbm.at[pl.ds((dirn * NCH2 + cc) * ACC_ROWS + sid * ZROWS,
                             ZROWS)])
        plsc.subcore_barrier()
        return pcarry

    lax.fori_loop(0, NPH, phase, 0)


def kernel(feat, user_ids, item_ids, edge_src, edge_dst,
           W_src, b_src, W_dst, b_dst):
    user_ids = user_ids.astype(jnp.int32)
    item_ids = item_ids.astype(jnp.int32)
    edge_src = edge_src.astype(jnp.int32)
    edge_dst = edge_dst.astype(jnp.int32)

    pad = jnp.zeros((NPAD - N_USERS,), jnp.int32)
    ids_all = jnp.concatenate([user_ids, pad, item_ids, pad])

    h_all = _node_gather(feat, ids_all)
    logits = _edge_logits(h_all, edge_src, edge_dst)
    alpha = _softmax(logits)

    fsrc = _transform_chunks(h_all, W_src, b_src, 0).reshape(-1, CW2)
    fdst = _transform_chunks(h_all, W_dst, b_dst, NPAD // TR_BM).reshape(
        -1, CW2)
    table = jnp.concatenate([fsrc, fdst], axis=0)

    idsg = jnp.concatenate([edge_src, edge_dst])
    idss = jnp.concatenate([edge_dst, edge_src]).reshape(-1, 1, GB)
    zeros = jnp.zeros((ZROWS, CW2), jnp.float32)
    out = _aggregate(table, idsg, idss, alpha, zeros)

    parts = out.reshape(2, NCH2, ACC_ROWS, CW2)
    e_new_item = (parts[0, :, :N_ITEMS]
                  .transpose(1, 0, 2).reshape(N_ITEMS, OUT_FEATS))
    e_new_user = (parts[1, :, :N_USERS]
                  .transpose(1, 0, 2).reshape(N_USERS, OUT_FEATS))
    return jnp.concatenate([e_new_user, e_new_item], axis=0)


# speedup vs baseline: 1.4093x; 1.0028x over previous
import functools
import jax
import jax.numpy as jnp
from jax import lax
from jax.experimental import pallas as pl
from jax.experimental.pallas import tpu as pltpu
from jax.experimental.pallas import tpu_sc as plsc

IN_FEATS = 256
OUT_FEATS = 256
N_USERS = 25000
N_ITEMS = 25000
N_EDGES = 1600000

NC, NS, L = 2, 16, 16
NW = NC * NS
NPAD = 25088
GB = 80
ACC_ROWS = 25600
FCH = 4
CW = OUT_FEATS // FCH

_mesh = plsc.VectorSubcoreMesh(
    core_axis_name="c", subcore_axis_name="s", num_cores=NC, num_subcores=NS)


NG_ROWS = 2 * NPAD
NG_PW = NG_ROWS // NW
NG_BLK = 112


@functools.partial(
    pl.kernel,
    out_type=jax.ShapeDtypeStruct((NG_ROWS, IN_FEATS), jnp.float32),
    mesh=_mesh,
    compiler_params=pltpu.CompilerParams(use_tc_tiling_on_sc=False,
                                         needs_layout_passes=False),
    scratch_types=[
        pltpu.VMEM((NG_BLK,), jnp.int32),
        pltpu.VMEM((NG_BLK, IN_FEATS), jnp.float32),
        pltpu.SemaphoreType.DMA,
    ],
)
def _node_gather(feat_hbm, ids_hbm, out_hbm, idx_v, rows_v, sem):
    wid = lax.axis_index("s") * NC + lax.axis_index("c")
    base = wid * NG_PW
    for k in range(NG_PW // NG_BLK):
        off = base + k * NG_BLK
        pltpu.sync_copy(ids_hbm.at[pl.ds(off, NG_BLK)], idx_v)
        pltpu.async_copy(feat_hbm.at[idx_v], rows_v, sem).wait()
        pltpu.sync_copy(rows_v, out_hbm.at[pl.ds(off, NG_BLK)])


EPW = N_EDGES // NW
LG_MEGA = 2000
LG_NSB = LG_MEGA // GB
LG_NMEGA = EPW // LG_MEGA


@functools.partial(
    pl.kernel,
    out_type=jax.ShapeDtypeStruct((N_EDGES,), jnp.float32),
    mesh=_mesh,
    compiler_params=pltpu.CompilerParams(use_tc_tiling_on_sc=False,
                                         needs_layout_passes=False),
    scratch_types=[
        pltpu.VMEM((LG_MEGA,), jnp.int32),
        pltpu.VMEM((LG_MEGA,), jnp.int32),
        pltpu.VMEM((LG_MEGA,), jnp.float32),
        pltpu.VMEM((2, GB, IN_FEATS), jnp.float32),
        pltpu.VMEM((2, GB, IN_FEATS), jnp.float32),
        pltpu.SemaphoreType.DMA,
        pltpu.SemaphoreType.DMA,
        pltpu.SemaphoreType.DMA,
        pltpu.SemaphoreType.DMA,
        pltpu.SemaphoreType.DMA,
        pltpu.SemaphoreType.DMA,
    ],
)
def _edge_logits(h_hbm, esrc_hbm, edst_hbm, out_hbm,
                 es_v, ed_v, lg_v, u_v, v_v, su0, su1, sv0, sv1, sh0, sh1):
    wid = lax.axis_index("s") * NC + lax.axis_index("c")
    base = wid * EPW
    sem_u = (su0, su1)
    sem_v = (sv0, sv1)

    def fire(k, slot):
        cu = pltpu.async_copy(
            h_hbm.at[es_v.at[pl.ds(k * GB, GB)]], u_v.at[slot], sem_u[slot])
        cv = pltpu.async_copy(
            h_hbm.at[ed_v.at[pl.ds(k * GB, GB)]], v_v.at[slot], sem_v[slot])
        return cu, cv

    def mega(m, carry):
        off = base + m * LG_MEGA
        h1 = pltpu.async_copy(esrc_hbm.at[pl.ds(off, LG_MEGA)], es_v, sh0)
        h2 = pltpu.async_copy(edst_hbm.at[pl.ds(off, LG_MEGA)], ed_v, sh1)
        h1.wait()
        h2.wait()

        def obody(q, c2):
            ed_v[pl.ds(q * L, L)] = ed_v[pl.ds(q * L, L)] + NPAD
            return c2
        lax.fori_loop(0, LG_MEGA // L, obody, 0)

        pend = [None, None]
        pend[0] = fire(0, 0)
        for k in range(LG_NSB):
            slot = k % 2
            if k + 1 < LG_NSB:
                pend[1 - slot] = fire(k + 1, 1 - slot)
            cu, cv = pend[slot]
            cu.wait()
            cv.wait()

            iot = lax.iota(jnp.int32, L)

            def group(g, c3):
                def jbody(j, res):
                    e = g * L + j
                    accs = []
                    for a0 in range(4):
                        c0 = a0 * 4
                        acc = (u_v[slot, e, pl.ds(c0 * L, L)]
                               * v_v[slot, e, pl.ds(c0 * L, L)])
                        for i in range(1, 4):
                            acc = acc + (u_v[slot, e, pl.ds((c0 + i) * L, L)]
                                         * v_v[slot, e, pl.ds((c0 + i) * L, L)])
                        accs.append(acc)
                    tot = (accs[0] + accs[1]) + (accs[2] + accs[3])
                    s = jnp.sum(tot) * 0.0625
                    return jnp.where(iot == j, s, res)

                res = lax.fori_loop(0, L, jbody, jnp.zeros((L,), jnp.float32))
                lg_v[pl.ds(k * GB + g * L, L)] = res
                return c3

            lax.fori_loop(0, GB // L, group, 0)
        pltpu.sync_copy(lg_v, out_hbm.at[pl.ds(off, LG_MEGA)])
        return carry

    lax.fori_loop(0, LG_NMEGA, mega, 0)


SM_COLS = 128
SM_ROWS = N_EDGES // SM_COLS
SM_BM = 500


def _sm_max_body(x_ref, o_ref):
    @pl.when(pl.program_id(0) == 0)
    def _():
        o_ref[0, 0] = -jnp.inf

    o_ref[0, 0] = jnp.maximum(o_ref[0, 0], jnp.max(x_ref[...]))


def _sm_exp_body(x_ref, m_ref, w_ref, z_ref):
    @pl.when(pl.program_id(0) == 0)
    def _():
        z_ref[0, 0] = 0.0

    w = jnp.exp(x_ref[...] - m_ref[0, 0])
    w_ref[...] = w
    z_ref[0, 0] += jnp.sum(w)


def _sm_scale_body(w_ref, z_ref, a_ref):
    a_ref[...] = w_ref[...] * (1.0 / z_ref[0, 0])


def _softmax(logits):
    x = logits.reshape(SM_ROWS, SM_COLS)
    m = pl.pallas_call(
        _sm_max_body,
        grid=(1,),
        in_specs=[pl.BlockSpec((SM_ROWS, SM_COLS), lambda i: (0, 0))],
        out_specs=pl.BlockSpec(memory_space=pltpu.SMEM),
        out_shape=jax.ShapeDtypeStruct((1, 1), jnp.float32),
    )(x)
    w, z = pl.pallas_call(
        _sm_exp_body,
        grid=(1,),
        in_specs=[
            pl.BlockSpec((SM_ROWS, SM_COLS), lambda i: (0, 0)),
            pl.BlockSpec(memory_space=pltpu.SMEM),
        ],
        out_specs=[
            pl.BlockSpec((SM_ROWS, SM_COLS), lambda i: (0, 0)),
            pl.BlockSpec(memory_space=pltpu.SMEM),
        ],
        out_shape=[
            jax.ShapeDtypeStruct((SM_ROWS, SM_COLS), jnp.float32),
            jax.ShapeDtypeStruct((1, 1), jnp.float32),
        ],
    )(x, m)
    a = pl.pallas_call(
        _sm_scale_body,
        grid=(1,),
        in_specs=[
            pl.BlockSpec((SM_ROWS, SM_COLS), lambda i: (0, 0)),
            pl.BlockSpec(memory_space=pltpu.SMEM),
        ],
        out_specs=pl.BlockSpec((SM_ROWS, SM_COLS), lambda i: (0, 0)),
        out_shape=jax.ShapeDtypeStruct((SM_ROWS, SM_COLS), jnp.float32),
    )(w, z)
    return a.reshape(N_EDGES)


TR_BM = 784


def _tr_body(h_ref, w_ref, b_ref, o_ref):
    o_ref[0] = jax.nn.relu(
        lax.dot_general(h_ref[...], w_ref[0], (((1,), (1,)), ((), ())),
                        preferred_element_type=jnp.float32) + b_ref[0])


def _transform_chunks(h_all, W, b, blk_off):
    nb = NPAD // TR_BM
    return pl.pallas_call(
        _tr_body,
        grid=(nb, FCH),
        in_specs=[
            pl.BlockSpec((TR_BM, IN_FEATS), lambda i, c: (i + blk_off, 0)),
            pl.BlockSpec((1, CW, IN_FEATS), lambda i, c: (c, 0, 0)),
            pl.BlockSpec((1, 1, CW), lambda i, c: (c, 0, 0)),
        ],
        out_specs=pl.BlockSpec((1, TR_BM, CW), lambda i, c: (c, i, 0)),
        out_shape=jax.ShapeDtypeStruct((FCH, NPAD, CW), jnp.float32),
    )(h_all, W.reshape(FCH, CW, IN_FEATS), b.reshape(FCH, 1, CW))


EPS = N_EDGES // NS
ZROWS = ACC_ROWS // NS
AG_MEGA = 4000
AG_NSB = AG_MEGA // GB
AG_NMEGA = EPS // AG_MEGA
NSLOT = 8
AG_PF = 3
CW2 = 32
NCH2 = OUT_FEATS // CW2
NPH = 2 * (NCH2 // NC)


@functools.partial(
    pl.kernel,
    out_type=jax.ShapeDtypeStruct((2 * NCH2 * ACC_ROWS, CW2), jnp.float32),
    mesh=_mesh,
    compiler_params=pltpu.CompilerParams(use_tc_tiling_on_sc=False,
                                         needs_layout_passes=False),
    scratch_types=[
        pltpu.VMEM((AG_MEGA,), jnp.int32),
        pltpu.VMEM((AG_NSB, 1, GB), jnp.int32),
        pltpu.VMEM((AG_MEGA + L,), jnp.float32),
        pltpu.VMEM((NSLOT, GB, CW2), jnp.float32),
        pltpu.VMEM_SHARED((ACC_ROWS, CW2), jnp.float32),
    ] + [pltpu.SemaphoreType.DMA] * (2 * NSLOT + 3),
)
def _aggregate(table_hbm, idsg_hbm, idss_hbm, alpha_hbm, zeros_hbm,
               out_hbm, gi_v, si_v, al_v, rows_v, acc_sh, *sems):
    cid = lax.axis_index("c")
    sid = lax.axis_index("s")
    ebase = sid * EPS
    sem_g = sems[:NSLOT]
    sem_s = sems[NSLOT:2 * NSLOT]
    sem_sc = sems[2 * NSLOT:]

    def phase(p, pcarry):
        dirn = p // (NPH // 2)
        lchunk = p % (NPH // 2)
        cc = cid * (NPH // 2) + lchunk
        goffc = (cc % 2) + (dirn * (NCH2 // 2) + cc // 2) * (2 * NPAD)

        pltpu.sync_copy(zeros_hbm, acc_sh.at[pl.ds(sid * ZROWS, ZROWS)])
        plsc.subcore_barrier()

        def mega(m, carry):
            off = ebase + m * AG_MEGA
            c1 = pltpu.async_copy(
                idsg_hbm.at[pl.ds(dirn * N_EDGES + off, AG_MEGA)], gi_v,
                sem_sc[0])
            c2 = pltpu.async_copy(
                idss_hbm.at[pl.ds(dirn * (N_EDGES // GB) + off // GB, AG_NSB)],
                si_v, sem_sc[1])
            c3 = pltpu.async_copy(alpha_hbm.at[pl.ds(off, AG_MEGA)],
                                  al_v.at[pl.ds(0, AG_MEGA)], sem_sc[2])
            c1.wait()
            c2.wait()
            c3.wait()

            def obody(q, c2):
                gi_v[pl.ds(q * L, L)] = gi_v[pl.ds(q * L, L)] * 2 + goffc
                return c2
            lax.fori_loop(0, AG_MEGA // L, obody, 0)

            pend_g = [None] * NSLOT
            pend_s = [None] * NSLOT

            def fire(k):
                slot = k % NSLOT
                pend_g[slot] = pltpu.async_copy(
                    table_hbm.at[gi_v.at[pl.ds(k * GB, GB)]],
                    rows_v.at[slot], sem_g[slot])

            for kp in range(AG_PF):
                fire(kp)
            for k in range(AG_NSB):
                slot = k % NSLOT
                if k + AG_PF < AG_NSB:
                    nslot = (k + AG_PF) % NSLOT
                    if pend_s[nslot] is not None:
                        pend_s[nslot].wait()
                        pend_s[nslot] = None
                    fire(k + AG_PF)
                pend_g[slot].wait()
                rows = rows_v.at[slot]

                def ebody(gq, c3):
                    a16 = al_v[pl.ds(k * GB + gq * L, L)]
                    for j in range(L):
                        e = gq * L + j
                        av = jnp.full((L,), a16[j])
                        for c2 in range(CW2 // L):
                            sl = pl.ds(c2 * L, L)
                            rows_v[slot, e, sl] = rows_v[slot, e, sl] * av
                    return c3

                lax.fori_loop(0, GB // L, ebody, 0)
                pend_s[slot] = pltpu.async_copy(
                    rows, acc_sh.at[si_v.at[k, 0]], sem_s[slot], add=True)
            for pd in pend_s:
                if pd is not None:
                    pd.wait()
            return carry

        lax.fori_loop(0, AG_NMEGA, mega, 0)
        plsc.subcore_barrier()
        pltpu.sync_copy(
            acc_sh.at[pl.ds(sid * ZROWS, ZROWS)],
            out_hbm.at[pl.ds((dirn * NCH2 + cc) * ACC_ROWS + sid * ZROWS,
                             ZROWS)])
        plsc.subcore_barrier()
        return pcarry

    lax.fori_loop(0, NPH, phase, 0)


def kernel(feat, user_ids, item_ids, edge_src, edge_dst,
           W_src, b_src, W_dst, b_dst):
    user_ids = user_ids.astype(jnp.int32)
    item_ids = item_ids.astype(jnp.int32)
    edge_src = edge_src.astype(jnp.int32)
    edge_dst = edge_dst.astype(jnp.int32)

    pad = jnp.zeros((NPAD - N_USERS,), jnp.int32)
    ids_all = jnp.concatenate([user_ids, pad, item_ids, pad])

    h_all = _node_gather(feat, ids_all)
    logits = _edge_logits(h_all, edge_src, edge_dst)
    alpha = _softmax(logits)

    fsrc = _transform_chunks(h_all, W_src, b_src, 0).reshape(-1, CW2)
    fdst = _transform_chunks(h_all, W_dst, b_dst, NPAD // TR_BM).reshape(
        -1, CW2)
    table = jnp.concatenate([fsrc, fdst], axis=0)

    idsg = jnp.concatenate([edge_src, edge_dst])
    idss = jnp.concatenate([edge_dst, edge_src]).reshape(-1, 1, GB)
    zeros = jnp.zeros((ZROWS, CW2), jnp.float32)
    out = _aggregate(table, idsg, idss, alpha, zeros)

    parts = out.reshape(2, NCH2, ACC_ROWS, CW2)
    e_new_item = (parts[0, :, :N_ITEMS]
                  .transpose(1, 0, 2).reshape(N_ITEMS, OUT_FEATS))
    e_new_user = (parts[1, :, :N_USERS]
                  .transpose(1, 0, 2).reshape(N_USERS, OUT_FEATS))
    return jnp.concatenate([e_new_user, e_new_item], axis=0)
